# baseline (device time: 63215 ns/iter reference)
import os

import jax
import jax.numpy as jnp
from jax import lax
from jax.experimental import pallas as pl
from jax.experimental.pallas import tpu as pltpu

N_DEV = 32
M_PER = 128

K_CHUNK = int(os.environ.get("K_CHUNK", "512"))
W_SLOTS = int(os.environ.get("W_SLOTS", "2"))
N_SPLIT = int(os.environ.get("N_SPLIT", "2"))

_VARIANT = os.environ.get("KERNEL_VARIANT", "full")


def kernel(x, w_mat, scale_x, scale_w):
    m, k_shard = x.shape
    k, n = w_mat.shape
    m_per = m // N_DEV
    n_chunks = k // K_CHUNK
    bpc = K_CHUNK // k_shard
    n_sub = n // N_SPLIT

    def body(x_ref, w_ref, sx_ref, sw_ref, out_ref,
             xq_ref, xrows_ref, wbuf_ref, send_sems, recv_sems, wdma_sems):
        me = lax.axis_index("i")

        xq_ref[:, :] = x_ref[:, :].astype(jnp.float8_e4m3fn)

        for j in range(N_DEV):
            @pl.when(j != me)
            def _(j=j):
                rdma = pltpu.make_async_remote_copy(
                    src_ref=xq_ref.at[pl.ds(j * m_per, m_per), :],
                    dst_ref=xrows_ref.at[me],
                    send_sem=send_sems.at[j],
                    recv_sem=recv_sems.at[me],
                    device_id=(j,),
                    device_id_type=pl.DeviceIdType.MESH,
                )
                rdma.start()

        xrows_ref[me] = xq_ref[pl.ds(me * m_per, m_per), :]

        def w_dma(c, slot, s):
            return pltpu.make_async_copy(
                w_ref.at[pl.ds(c * K_CHUNK, K_CHUNK), pl.ds(s * n_sub, n_sub)],
                wbuf_ref.at[slot, :, pl.ds(s * n_sub, n_sub)],
                wdma_sems.at[slot, s],
            )

        for c in range(min(W_SLOTS, n_chunks)):
            for s in range(N_SPLIT):
                w_dma(c, c, s).start()

        out_ref[:, :] = jnp.zeros((m_per, n), dtype=jnp.float32)

        for c in range(n_chunks):
            for b in range(bpc):
                j = c * bpc + b

                @pl.when(j != me)
                def _(j=j):
                    recv = pltpu.make_async_remote_copy(
                        src_ref=xq_ref.at[pl.ds(0, m_per), :],
                        dst_ref=xrows_ref.at[j],
                        send_sem=send_sems.at[j],
                        recv_sem=recv_sems.at[j],
                        device_id=(j,),
                        device_id_type=pl.DeviceIdType.MESH,
                    )
                    recv.wait_recv()

            slot = c % W_SLOTS
            for s in range(N_SPLIT):
                w_dma(c, slot, s).wait()

            if _VARIANT == "full":
                lhs = jnp.concatenate(
                    [xrows_ref[c * bpc + b] for b in range(bpc)], axis=1
                )
                wq = wbuf_ref[slot].astype(jnp.float8_e5m2)
                out_ref[:, :] += jnp.dot(
                    lhs, wq, preferred_element_type=jnp.float32,
                )

            if c + W_SLOTS < n_chunks:
                for s in range(N_SPLIT):
                    w_dma(c + W_SLOTS, slot, s).start()

        scale = sx_ref[0] * sw_ref[0]
        out_ref[:, :] = jnp.maximum(out_ref[:, :] * scale, 0.0)

        for j in range(N_DEV):
            @pl.when(j != me)
            def _(j=j):
                send = pltpu.make_async_remote_copy(
                    src_ref=xq_ref.at[pl.ds(j * m_per, m_per), :],
                    dst_ref=xrows_ref.at[me],
                    send_sem=send_sems.at[j],
                    recv_sem=recv_sems.at[me],
                    device_id=(j,),
                    device_id_type=pl.DeviceIdType.MESH,
                )
                send.wait_send()

    return pl.pallas_call(
        body,
        out_shape=jax.ShapeDtypeStruct((m_per, n), jnp.float32),
        in_specs=[
            pl.BlockSpec(memory_space=pltpu.VMEM),
            pl.BlockSpec(memory_space=pl.ANY),
            pl.BlockSpec(memory_space=pltpu.SMEM),
            pl.BlockSpec(memory_space=pltpu.SMEM),
        ],
        out_specs=pl.BlockSpec(memory_space=pltpu.VMEM),
        scratch_shapes=[
            pltpu.VMEM((m, k_shard), jnp.float8_e4m3fn),
            pltpu.VMEM((N_DEV, m_per, k_shard), jnp.float8_e4m3fn),
            pltpu.VMEM((W_SLOTS, K_CHUNK, n), jnp.float32),
            pltpu.SemaphoreType.DMA((N_DEV,)),
            pltpu.SemaphoreType.DMA((N_DEV,)),
            pltpu.SemaphoreType.DMA((W_SLOTS, N_SPLIT)),
        ],
        compiler_params=pltpu.CompilerParams(
            vmem_limit_bytes=56 * 1024 * 1024,
        ),
    )(x, w_mat, scale_x, scale_w)
